# baseline (speedup 1.0000x reference)
import jax
import jax.numpy as jnp
from jax.experimental import pallas as pl
from jax.experimental.pallas import tpu as pltpu

_NUM_EXPERTS = 16
_TOP_K = 2
_BLOCK_ROWS = 2048
_K_CHUNK = 512


def _gate_kernel(x_ref, w_ref, idx_ref, val_ref, acc_ref):
    k = pl.program_id(1)
    nk = pl.num_programs(1)

    x = x_ref[...]
    w = w_ref[...]
    part = jax.lax.dot_general(
        x, w, (((1,), (1,)), ((), ())), preferred_element_type=jnp.float32
    )

    @pl.when(k == 0)
    def _init():
        acc_ref[...] = part

    @pl.when(k != 0)
    def _accum():
        acc_ref[...] += part

    @pl.when(k == nk - 1)
    def _select():
        logits = acc_ref[...]
        col = jax.lax.broadcasted_iota(jnp.int32, logits.shape, 1)

        m1 = jnp.max(logits, axis=1, keepdims=True)
        i1 = jnp.min(jnp.where(logits == m1, col, _NUM_EXPERTS), axis=1)

        masked = jnp.where(col == i1[:, None], -jnp.inf, logits)
        m2 = jnp.max(masked, axis=1, keepdims=True)
        i2 = jnp.min(jnp.where(masked == m2, col, _NUM_EXPERTS), axis=1)

        ex = jnp.exp(logits - m1)
        denom = jnp.sum(ex, axis=1, keepdims=True)
        v1 = 1.0 / denom[:, 0]
        v2 = jnp.exp(m2 - m1)[:, 0] / denom[:, 0]

        idx_ref[...] = jnp.concatenate([i1[:, None], i2[:, None]], axis=1)
        val_ref[...] = jnp.concatenate([v1[:, None], v2[:, None]], axis=1)


@jax.jit
def kernel(hidden_states, weight):
    d = hidden_states.shape[-1]
    hs = hidden_states.reshape(-1, d)
    t = hs.shape[0]
    grid = (t // _BLOCK_ROWS, d // _K_CHUNK)

    idx, val = pl.pallas_call(
        _gate_kernel,
        grid=grid,
        in_specs=[
            pl.BlockSpec((_BLOCK_ROWS, _K_CHUNK), lambda r, k: (r, k)),
            pl.BlockSpec((_NUM_EXPERTS, _K_CHUNK), lambda r, k: (0, k)),
        ],
        out_specs=[
            pl.BlockSpec((_BLOCK_ROWS, _TOP_K), lambda r, k: (r, 0)),
            pl.BlockSpec((_BLOCK_ROWS, _TOP_K), lambda r, k: (r, 0)),
        ],
        out_shape=[
            jax.ShapeDtypeStruct((t, _TOP_K), jnp.int32),
            jax.ShapeDtypeStruct((t, _TOP_K), jnp.float32),
        ],
        scratch_shapes=[pltpu.VMEM((_BLOCK_ROWS, _NUM_EXPERTS), jnp.float32)],
    )(hs, weight)
    return idx, val


# speedup vs baseline: 1.2756x; 1.2756x over previous
import jax
import jax.numpy as jnp
from jax.experimental import pallas as pl

_NUM_EXPERTS = 16
_TOP_K = 2
_BLOCK_ROWS = 512


def _select_top2(logits, idx_ref, val_ref):
    col = jax.lax.broadcasted_iota(jnp.int32, logits.shape, 1)
    revf = (15 - col).astype(jnp.float32)

    m1 = jnp.max(logits, axis=1, keepdims=True)
    r1 = jnp.max(jnp.where(logits == m1, revf, -1.0), axis=1, keepdims=True)

    masked = jnp.where(revf == r1, -jnp.inf, logits)
    m2 = jnp.max(masked, axis=1, keepdims=True)
    r2 = jnp.max(jnp.where(masked == m2, revf, -1.0), axis=1, keepdims=True)

    ex = jnp.exp(logits - m1)
    denom = jnp.sum(ex, axis=1, keepdims=True)
    v1 = 1.0 / denom
    v2 = jnp.exp(m2 - m1) / denom

    i1 = (15.0 - r1).astype(jnp.int32)
    i2 = (15.0 - r2).astype(jnp.int32)
    idx_ref[...] = jnp.concatenate([i1, i2], axis=1)
    val_ref[...] = jnp.concatenate([v1, v2], axis=1)


def _gate_kernel(x1_ref, x2_ref, w_ref, idx1_ref, val1_ref, idx2_ref, val2_ref):
    w = w_ref[...]
    dims = (((1,), (1,)), ((), ()))
    logits1 = jax.lax.dot_general(
        x1_ref[...], w, dims, preferred_element_type=jnp.float32
    )
    _select_top2(logits1, idx1_ref, val1_ref)
    logits2 = jax.lax.dot_general(
        x2_ref[...], w, dims, preferred_element_type=jnp.float32
    )
    _select_top2(logits2, idx2_ref, val2_ref)


@jax.jit
def kernel(hidden_states, weight):
    d = hidden_states.shape[-1]
    hs = hidden_states.reshape(-1, d)
    t = hs.shape[0]
    half = t // 2
    nblk = half // _BLOCK_ROWS
    grid = (nblk,)

    idx1, val1, idx2, val2 = pl.pallas_call(
        _gate_kernel,
        grid=grid,
        in_specs=[
            pl.BlockSpec((_BLOCK_ROWS, d), lambda i: (i, 0)),
            pl.BlockSpec((_BLOCK_ROWS, d), lambda i, nb=nblk: (i + nb, 0)),
            pl.BlockSpec((_NUM_EXPERTS, d), lambda i: (0, 0)),
        ],
        out_specs=[
            pl.BlockSpec((_BLOCK_ROWS, _TOP_K), lambda i: (i, 0)),
            pl.BlockSpec((_BLOCK_ROWS, _TOP_K), lambda i: (i, 0)),
            pl.BlockSpec((_BLOCK_ROWS, _TOP_K), lambda i: (i, 0)),
            pl.BlockSpec((_BLOCK_ROWS, _TOP_K), lambda i: (i, 0)),
        ],
        out_shape=[
            jax.ShapeDtypeStruct((half, _TOP_K), jnp.int32),
            jax.ShapeDtypeStruct((half, _TOP_K), jnp.float32),
            jax.ShapeDtypeStruct((half, _TOP_K), jnp.int32),
            jax.ShapeDtypeStruct((half, _TOP_K), jnp.float32),
        ],
    )(hs, hs, weight)
    idx = jnp.concatenate([idx1, idx2], axis=0)
    val = jnp.concatenate([val1, val2], axis=0)
    return idx, val
